# Initial kernel scaffold; baseline (speedup 1.0000x reference)
import jax
import jax.numpy as jnp
from jax.experimental import pallas as pl


def kernel(x, edge_index, drug_pos_ids, target_pos_ids, drug_neg_ids, target_neg_ids, adjacency_matrix, gcn_W, gcn_b, W_D, W_T, q_D, q_T, bn_gamma, bn_beta, out_W, out_b):
    raise NotImplementedError("write your pallas kernel here")


# speedup vs baseline: 18.1927x; 18.1927x over previous
import functools
import jax
import jax.numpy as jnp
from jax import lax
from jax.experimental import pallas as pl
from jax.experimental.pallas import tpu as pltpu
from jax.experimental.pallas import tpu_sc as plsc

NN = 10000
EE = 320000
HH = 128
NC = 2
NS = 16
NW = NC * NS
CHUNK = 128
NCH = 80
NPAD = 10240
RPT = NPAD // NS
KW = 8
NWIN = NCH // KW
_HIGH = lax.Precision.DEFAULT


@functools.cache
def _sc_kernels():
  mesh = plsc.VectorSubcoreMesh(
      core_axis_name="c", subcore_axis_name="s", num_cores=NC, num_subcores=NS)

  @functools.partial(
      pl.kernel,
      out_type=jax.ShapeDtypeStruct((NC, NPAD, HH), jnp.float32),
      mesh=mesh,
      scratch_types=[
          pltpu.VMEM((NCH, CHUNK), jnp.int32),
          pltpu.VMEM((CHUNK, HH), jnp.float32),
          pltpu.VMEM_SHARED((NPAD, HH), jnp.float32),
      ],
  )
  def sc_degree(dst_hbm, zrow_hbm, ones_hbm, out_hbm, dst_v, ones_v, deg_sp):
    c = lax.axis_index("c")
    s = lax.axis_index("s")
    w = c * NS + s
    pltpu.sync_copy(zrow_hbm, ones_v)
    for k in range(RPT // CHUNK):
      pltpu.sync_copy(ones_v, deg_sp.at[pl.ds((s * 5 + k) * CHUNK, CHUNK)])
    pltpu.sync_copy(dst_hbm.at[w], dst_v)
    pltpu.sync_copy(ones_hbm, ones_v)
    plsc.subcore_barrier()

    @pl.loop(0, NCH)
    def _(j):
      pltpu.sync_copy(ones_v, deg_sp.at[dst_v.at[j]], add=True)

    plsc.subcore_barrier()
    pltpu.sync_copy(deg_sp.at[pl.ds(s * RPT, RPT)],
                    out_hbm.at[c].at[pl.ds(s * RPT, RPT)])

  @functools.partial(
      pl.kernel,
      out_type=jax.ShapeDtypeStruct((NC, NPAD, HH), jnp.float32),
      mesh=mesh,
      scratch_types=[
          pltpu.VMEM((KW, CHUNK), jnp.int32),
          pltpu.VMEM((KW, CHUNK), jnp.int32),
          pltpu.VMEM((2, CHUNK, HH), jnp.float32),
          pltpu.VMEM_SHARED((NPAD, HH), jnp.float32),
          pltpu.SemaphoreType.DMA,
      ],
  )
  def sc_aggregate(scaled_hbm, src_hbm, dst_hbm, zrow_hbm, out_hbm,
                   sidx, didx, rows_v, agg_sp, sem0):
    c = lax.axis_index("c")
    s = lax.axis_index("s")
    w = c * NS + s
    pltpu.sync_copy(zrow_hbm, rows_v.at[0])
    for k in range(RPT // CHUNK):
      pltpu.sync_copy(rows_v.at[0], agg_sp.at[pl.ds((s * 5 + k) * CHUNK, CHUNK)])
    plsc.subcore_barrier()

    @pl.loop(0, NWIN)
    def _(wi):
      pltpu.sync_copy(src_hbm.at[w].at[pl.ds(wi * KW, KW)], sidx)
      pltpu.sync_copy(dst_hbm.at[w].at[pl.ds(wi * KW, KW)], didx)
      pltpu.async_copy(scaled_hbm.at[sidx.at[0]], rows_v.at[0], sem0)
      for b in range(KW):
        pltpu.make_async_copy(scaled_hbm.at[sidx.at[b]],
                              rows_v.at[b % 2], sem0).wait()
        if b + 1 < KW:
          pltpu.async_copy(scaled_hbm.at[sidx.at[b + 1]],
                           rows_v.at[(b + 1) % 2], sem0)
        pltpu.sync_copy(rows_v.at[b % 2], agg_sp.at[didx.at[b]], add=True)

    plsc.subcore_barrier()
    pltpu.sync_copy(agg_sp.at[pl.ds(s * RPT, RPT)],
                    out_hbm.at[c].at[pl.ds(s * RPT, RPT)])

  @functools.partial(
      pl.kernel,
      out_type=jax.ShapeDtypeStruct((4 * 8192, HH), jnp.float32),
      mesh=mesh,
      scratch_types=[
          pltpu.VMEM((8, CHUNK), jnp.int32),
          pltpu.VMEM((CHUNK, HH), jnp.float32),
          pltpu.SemaphoreType.DMA,
      ],
  )
  def sc_take(table_hbm, idx_hbm, out_hbm, idx_v, rows_v, sem):
    c = lax.axis_index("c")
    s = lax.axis_index("s")
    w = c * NS + s
    pltpu.sync_copy(idx_hbm.at[w], idx_v)

    @pl.loop(0, 8)
    def _(j):
      pltpu.async_copy(table_hbm.at[idx_v.at[j]], rows_v, sem).wait()
      pltpu.sync_copy(rows_v, out_hbm.at[pl.ds((w * 8 + j) * CHUNK, CHUNK)])

  return sc_degree, sc_aggregate, sc_take


RB = 2000
GB = NN // RB


def _tc_prep_body(x_ref, w1_ref, degp_ref, scaled_ref, dinv_ref):
  deg = degp_ref[0][:, 0:1] + degp_ref[1][:, 0:1] + 1.0
  dinv = lax.rsqrt(jnp.maximum(deg, 1.0))
  dinv_bc = jnp.broadcast_to(dinv, (RB, HH))
  hw = lax.dot_general(x_ref[...], w1_ref[...],
                       (((1,), (1,)), ((), ())), precision=_HIGH)
  scaled_ref[...] = hw * dinv_bc
  dinv_ref[...] = dinv_bc


def _tc_stats_body(parts_ref, scaled_ref, dinv_ref, b_ref,
                   h_ref, stats_ref, acc_ref):
  i = pl.program_id(0)
  agg = parts_ref[0] + parts_ref[1] + scaled_ref[...]
  h = jnp.maximum(agg * dinv_ref[...] + b_ref[...], 0.0)
  h_ref[...] = h
  ssum = jnp.sum(h, axis=0, keepdims=True)
  ssq = jnp.sum(jnp.square(h), axis=0, keepdims=True)
  blk = jnp.concatenate([ssum, ssq], axis=0)

  @pl.when(i == 0)
  def _():
    acc_ref[...] = blk

  @pl.when(i > 0)
  def _():
    acc_ref[...] += blk

  @pl.when(i == GB - 1)
  def _():
    stats_ref[...] = acc_ref[...]


def _tc_bn_body(h_ref, stats_ref, dinv_ref, g_ref, be_ref, wnext_ref,
                hbn_ref, snext_ref, *, last):
  mean = stats_ref[0:1] * (1.0 / NN)
  var = stats_ref[1:2] * (1.0 / NN) - jnp.square(mean)
  hbn = ((h_ref[...] - mean) * lax.rsqrt(var + 1e-5) * g_ref[...]
         + be_ref[...])
  hbn_ref[...] = hbn
  if not last:
    hw = lax.dot_general(hbn, wnext_ref[...],
                         (((1,), (1,)), ((), ())), precision=_HIGH)
    snext_ref[...] = hw * dinv_ref[...]


def _tc_attn_body(h0_ref, h1_ref, h2_ref, wd_ref, wt_ref, qd_ref, qt_ref,
                  ow_ref, ob_ref, zd_ref, zt_ref):
  hs = (h0_ref[...], h1_ref[...], h2_ref[...])

  def pool(w, q):
    es = []
    for h in hs:
      hh = lax.dot_general(h, w, (((1,), (1,)), ((), ())), precision=_HIGH)
      hh = jnp.where(hh >= 0.0, hh, 0.01 * hh)
      es.append(jnp.sum(hh * q, axis=1, keepdims=True))
    m = jnp.maximum(jnp.maximum(es[0], es[1]), es[2])
    a = [jnp.exp(e - m) for e in es]
    tot = a[0] + a[1] + a[2]
    pooled = (hs[0] * (a[0] / tot) + hs[1] * (a[1] / tot)
              + hs[2] * (a[2] / tot))
    return lax.dot_general(pooled, ow_ref[...], (((1,), (1,)), ((), ())),
                           precision=_HIGH) + ob_ref[...]

  zd_ref[...] = pool(wd_ref[...], qd_ref[...])
  zt_ref[...] = pool(wt_ref[...], qt_ref[...])


def _f32(shape):
  return jax.ShapeDtypeStruct(shape, jnp.float32)


def kernel(x, edge_index, drug_pos_ids, target_pos_ids, drug_neg_ids,
           target_neg_ids, adjacency_matrix, gcn_W, gcn_b, W_D, W_T, q_D, q_T,
           bn_gamma, bn_beta, out_W, out_b):
  sc_degree, sc_aggregate, sc_take = _sc_kernels()

  ept = EE // NW
  ppt = NCH * CHUNK
  padlen = ppt - ept
  src = edge_index[0].reshape(NW, ept)
  dst = edge_index[1].reshape(NW, ept)
  pad_iota = jnp.arange(NW * padlen, dtype=jnp.int32).reshape(NW, padlen)
  pad_src = pad_iota % NN
  pad_dst = NN + (pad_iota % (NPAD - NN))
  src_p = jnp.concatenate([src, pad_src], 1).reshape(NW, NCH, CHUNK)
  dst_p = jnp.concatenate([dst, pad_dst], 1).reshape(NW, NCH, CHUNK)

  zrow = jnp.zeros((CHUNK, HH), jnp.float32)
  onesrow = jnp.ones((CHUNK, HH), jnp.float32)

  rows = pl.BlockSpec((RB, HH), lambda i: (i, 0))
  full_w = pl.BlockSpec((HH, HH), lambda i: (0, 0))
  full_v = pl.BlockSpec((1, HH), lambda i: (0, 0))

  deg_parts = sc_degree(dst_p, zrow, onesrow)

  scaled, dinv_bc = pl.pallas_call(
      _tc_prep_body,
      grid=(GB,),
      in_specs=[rows, full_w,
                pl.BlockSpec((NC, RB, HH), lambda i: (0, i, 0))],
      out_specs=(rows, rows),
      out_shape=(_f32((NN, HH)), _f32((NN, HH))),
  )(x, gcn_W[0], deg_parts)

  hs = []
  for l in range(3):
    parts = sc_aggregate(scaled, src_p, dst_p, zrow)
    last = l == 2
    wnext = gcn_W[l + 1] if not last else jnp.zeros((HH, HH), jnp.float32)

    h_pre, stats = pl.pallas_call(
        _tc_stats_body,
        grid=(GB,),
        in_specs=[pl.BlockSpec((NC, RB, HH), lambda i: (0, i, 0)),
                  rows, rows, full_v],
        out_specs=(rows, pl.BlockSpec((2, HH), lambda i: (0, 0))),
        out_shape=(_f32((NN, HH)), _f32((2, HH))),
        scratch_shapes=[pltpu.VMEM((2, HH), jnp.float32)],
    )(parts, scaled, dinv_bc, gcn_b[l].reshape(1, HH))

    h, scaled = pl.pallas_call(
        functools.partial(_tc_bn_body, last=last),
        grid=(GB,),
        in_specs=[rows, pl.BlockSpec((2, HH), lambda i: (0, 0)),
                  rows, full_v, full_v, full_w],
        out_specs=(rows, rows),
        out_shape=(_f32((NN, HH)), _f32((NN, HH))),
    )(h_pre, stats, dinv_bc, bn_gamma.reshape(1, HH),
      bn_beta.reshape(1, HH), wnext)
    hs.append(h)

  z_D, z_T = pl.pallas_call(
      _tc_attn_body,
      grid=(GB,),
      in_specs=[rows, rows, rows, full_w, full_w, full_v, full_v,
                full_w, full_v],
      out_specs=(rows, rows),
      out_shape=(_f32((NN, HH)), _f32((NN, HH))),
  )(hs[0], hs[1], hs[2], W_D, W_T, q_D.reshape(1, HH), q_T.reshape(1, HH),
    out_W, out_b.reshape(1, HH))

  zcat = jnp.concatenate([z_D, z_T], axis=0)
  idx_all = jnp.concatenate([
      drug_pos_ids, target_pos_ids + NN, drug_neg_ids, target_neg_ids + NN,
  ]).reshape(NW, 8, CHUNK)
  gathered = sc_take(zcat, idx_all)
  g = gathered.reshape(4, 8192, HH)
  return (g[0], g[1], g[2], g[3])


# speedup vs baseline: 19.2040x; 1.0556x over previous
import functools
import jax
import jax.numpy as jnp
from jax import lax
from jax.experimental import pallas as pl
from jax.experimental.pallas import tpu as pltpu
from jax.experimental.pallas import tpu_sc as plsc

NN = 10000
EE = 320000
HH = 128
NC = 2
NS = 16
NW = NC * NS
CHUNK = 128
NCH = 80
NPAD = 10240
RPT = NPAD // NS
KW = 8
NWIN = NCH // KW
_HIGH = lax.Precision.DEFAULT


@functools.cache
def _sc_kernels():
  mesh = plsc.VectorSubcoreMesh(
      core_axis_name="c", subcore_axis_name="s", num_cores=NC, num_subcores=NS)

  @functools.partial(
      pl.kernel,
      out_type=jax.ShapeDtypeStruct((NC, NPAD, HH), jnp.float32),
      mesh=mesh,
      scratch_types=[
          pltpu.VMEM((NCH, CHUNK), jnp.int32),
          pltpu.VMEM((CHUNK, HH), jnp.float32),
          pltpu.VMEM_SHARED((NPAD, HH), jnp.float32),
      ],
  )
  def sc_degree(dst_hbm, zrow_hbm, ones_hbm, out_hbm, dst_v, ones_v, deg_sp):
    c = lax.axis_index("c")
    s = lax.axis_index("s")
    w = c * NS + s
    pltpu.sync_copy(zrow_hbm, ones_v)
    for k in range(RPT // CHUNK):
      pltpu.sync_copy(ones_v, deg_sp.at[pl.ds((s * 5 + k) * CHUNK, CHUNK)])
    pltpu.sync_copy(dst_hbm.at[w], dst_v)
    pltpu.sync_copy(ones_hbm, ones_v)
    plsc.subcore_barrier()

    @pl.loop(0, NCH)
    def _(j):
      pltpu.sync_copy(ones_v, deg_sp.at[dst_v.at[j]], add=True)

    plsc.subcore_barrier()
    pltpu.sync_copy(deg_sp.at[pl.ds(s * RPT, RPT)],
                    out_hbm.at[c].at[pl.ds(s * RPT, RPT)])

  @functools.partial(
      pl.kernel,
      out_type=jax.ShapeDtypeStruct((NC, NPAD, HH), jnp.float32),
      mesh=mesh,
      scratch_types=[
          pltpu.VMEM((2, KW, CHUNK), jnp.int32),
          pltpu.VMEM((2, KW, CHUNK), jnp.int32),
          pltpu.VMEM((2, CHUNK, HH), jnp.float32),
          pltpu.VMEM_SHARED((NPAD, HH), jnp.float32),
          pltpu.SemaphoreType.DMA,
          pltpu.SemaphoreType.DMA,
      ],
  )
  def sc_aggregate(scaled_hbm, src_hbm, dst_hbm, zrow_hbm, out_hbm,
                   sidx, didx, rows_v, agg_sp, sem_g, sem_i):
    c = lax.axis_index("c")
    s = lax.axis_index("s")
    w = c * NS + s
    pltpu.sync_copy(zrow_hbm, rows_v.at[0])
    for k in range(RPT // CHUNK):
      pltpu.sync_copy(rows_v.at[0], agg_sp.at[pl.ds((s * 5 + k) * CHUNK, CHUNK)])
    plsc.subcore_barrier()

    pltpu.sync_copy(src_hbm.at[w].at[pl.ds(0, KW)], sidx.at[0])
    pltpu.sync_copy(dst_hbm.at[w].at[pl.ds(0, KW)], didx.at[0])
    pltpu.async_copy(scaled_hbm.at[sidx.at[0].at[0]], rows_v.at[0], sem_g)
    for wi in range(NWIN):
      cur, nxt = wi % 2, 1 - wi % 2
      if wi + 1 < NWIN:
        pltpu.async_copy(src_hbm.at[w].at[pl.ds((wi + 1) * KW, KW)],
                         sidx.at[nxt], sem_i)
        pltpu.async_copy(dst_hbm.at[w].at[pl.ds((wi + 1) * KW, KW)],
                         didx.at[nxt], sem_i)
      for b in range(KW):
        k = wi * KW + b
        buf = k % 2
        pltpu.make_async_copy(scaled_hbm.at[sidx.at[cur].at[b]],
                              rows_v.at[buf], sem_g).wait()
        if b + 1 < KW:
          pltpu.async_copy(scaled_hbm.at[sidx.at[cur].at[b + 1]],
                           rows_v.at[1 - buf], sem_g)
        elif wi + 1 < NWIN:
          pltpu.make_async_copy(src_hbm.at[w].at[pl.ds((wi + 1) * KW, KW)],
                                sidx.at[nxt], sem_i).wait()
          pltpu.make_async_copy(dst_hbm.at[w].at[pl.ds((wi + 1) * KW, KW)],
                                didx.at[nxt], sem_i).wait()
          pltpu.async_copy(scaled_hbm.at[sidx.at[nxt].at[0]],
                           rows_v.at[1 - buf], sem_g)
        pltpu.sync_copy(rows_v.at[buf], agg_sp.at[didx.at[cur].at[b]],
                        add=True)

    plsc.subcore_barrier()
    pltpu.sync_copy(agg_sp.at[pl.ds(s * RPT, RPT)],
                    out_hbm.at[c].at[pl.ds(s * RPT, RPT)])

  @functools.partial(
      pl.kernel,
      out_type=jax.ShapeDtypeStruct((4 * 8192, HH), jnp.float32),
      mesh=mesh,
      scratch_types=[
          pltpu.VMEM((8, CHUNK), jnp.int32),
          pltpu.VMEM((CHUNK, HH), jnp.float32),
          pltpu.SemaphoreType.DMA,
      ],
  )
  def sc_take(table_hbm, idx_hbm, out_hbm, idx_v, rows_v, sem):
    c = lax.axis_index("c")
    s = lax.axis_index("s")
    w = c * NS + s
    pltpu.sync_copy(idx_hbm.at[w], idx_v)

    @pl.loop(0, 8)
    def _(j):
      pltpu.async_copy(table_hbm.at[idx_v.at[j]], rows_v, sem).wait()
      pltpu.sync_copy(rows_v, out_hbm.at[pl.ds((w * 8 + j) * CHUNK, CHUNK)])

  return sc_degree, sc_aggregate, sc_take


RB = 2000
GB = NN // RB


def _tc_prep_body(x_ref, w1_ref, degp_ref, scaled_ref, dinv_ref):
  deg = degp_ref[0][:, 0:1] + degp_ref[1][:, 0:1] + 1.0
  dinv = lax.rsqrt(jnp.maximum(deg, 1.0))
  dinv_bc = jnp.broadcast_to(dinv, (RB, HH))
  hw = lax.dot_general(x_ref[...], w1_ref[...],
                       (((1,), (1,)), ((), ())), precision=_HIGH)
  scaled_ref[...] = hw * dinv_bc
  dinv_ref[...] = dinv_bc


def _tc_stats_body(parts_ref, scaled_ref, dinv_ref, b_ref,
                   h_ref, stats_ref, acc_ref):
  i = pl.program_id(0)
  agg = parts_ref[0] + parts_ref[1] + scaled_ref[...]
  h = jnp.maximum(agg * dinv_ref[...] + b_ref[...], 0.0)
  h_ref[...] = h
  ssum = jnp.sum(h, axis=0, keepdims=True)
  ssq = jnp.sum(jnp.square(h), axis=0, keepdims=True)
  blk = jnp.concatenate([ssum, ssq], axis=0)

  @pl.when(i == 0)
  def _():
    acc_ref[...] = blk

  @pl.when(i > 0)
  def _():
    acc_ref[...] += blk

  @pl.when(i == GB - 1)
  def _():
    stats_ref[...] = acc_ref[...]


def _tc_bn_body(h_ref, stats_ref, dinv_ref, g_ref, be_ref, wnext_ref,
                hbn_ref, snext_ref, *, last):
  mean = stats_ref[0:1] * (1.0 / NN)
  var = stats_ref[1:2] * (1.0 / NN) - jnp.square(mean)
  hbn = ((h_ref[...] - mean) * lax.rsqrt(var + 1e-5) * g_ref[...]
         + be_ref[...])
  hbn_ref[...] = hbn
  if not last:
    hw = lax.dot_general(hbn, wnext_ref[...],
                         (((1,), (1,)), ((), ())), precision=_HIGH)
    snext_ref[...] = hw * dinv_ref[...]


def _tc_attn_body(h0_ref, h1_ref, h2_ref, wd_ref, wt_ref, qd_ref, qt_ref,
                  ow_ref, ob_ref, zd_ref, zt_ref):
  hs = (h0_ref[...], h1_ref[...], h2_ref[...])

  def pool(w, q):
    es = []
    for h in hs:
      hh = lax.dot_general(h, w, (((1,), (1,)), ((), ())), precision=_HIGH)
      hh = jnp.where(hh >= 0.0, hh, 0.01 * hh)
      es.append(jnp.sum(hh * q, axis=1, keepdims=True))
    m = jnp.maximum(jnp.maximum(es[0], es[1]), es[2])
    a = [jnp.exp(e - m) for e in es]
    tot = a[0] + a[1] + a[2]
    pooled = (hs[0] * (a[0] / tot) + hs[1] * (a[1] / tot)
              + hs[2] * (a[2] / tot))
    return lax.dot_general(pooled, ow_ref[...], (((1,), (1,)), ((), ())),
                           precision=_HIGH) + ob_ref[...]

  zd_ref[...] = pool(wd_ref[...], qd_ref[...])
  zt_ref[...] = pool(wt_ref[...], qt_ref[...])


def _f32(shape):
  return jax.ShapeDtypeStruct(shape, jnp.float32)


def kernel(x, edge_index, drug_pos_ids, target_pos_ids, drug_neg_ids,
           target_neg_ids, adjacency_matrix, gcn_W, gcn_b, W_D, W_T, q_D, q_T,
           bn_gamma, bn_beta, out_W, out_b):
  sc_degree, sc_aggregate, sc_take = _sc_kernels()

  ept = EE // NW
  ppt = NCH * CHUNK
  padlen = ppt - ept
  src = edge_index[0].reshape(NW, ept)
  dst = edge_index[1].reshape(NW, ept)
  pad_iota = jnp.arange(NW * padlen, dtype=jnp.int32).reshape(NW, padlen)
  pad_src = pad_iota % NN
  pad_dst = NN + (pad_iota % (NPAD - NN))
  src_p = jnp.concatenate([src, pad_src], 1).reshape(NW, NCH, CHUNK)
  dst_p = jnp.concatenate([dst, pad_dst], 1).reshape(NW, NCH, CHUNK)

  zrow = jnp.zeros((CHUNK, HH), jnp.float32)
  onesrow = jnp.ones((CHUNK, HH), jnp.float32)

  rows = pl.BlockSpec((RB, HH), lambda i: (i, 0))
  full_w = pl.BlockSpec((HH, HH), lambda i: (0, 0))
  full_v = pl.BlockSpec((1, HH), lambda i: (0, 0))

  deg_parts = sc_degree(dst_p, zrow, onesrow)

  scaled, dinv_bc = pl.pallas_call(
      _tc_prep_body,
      grid=(GB,),
      in_specs=[rows, full_w,
                pl.BlockSpec((NC, RB, HH), lambda i: (0, i, 0))],
      out_specs=(rows, rows),
      out_shape=(_f32((NN, HH)), _f32((NN, HH))),
  )(x, gcn_W[0], deg_parts)

  hs = []
  for l in range(3):
    parts = sc_aggregate(scaled, src_p, dst_p, zrow)
    last = l == 2
    wnext = gcn_W[l + 1] if not last else jnp.zeros((HH, HH), jnp.float32)

    h_pre, stats = pl.pallas_call(
        _tc_stats_body,
        grid=(GB,),
        in_specs=[pl.BlockSpec((NC, RB, HH), lambda i: (0, i, 0)),
                  rows, rows, full_v],
        out_specs=(rows, pl.BlockSpec((2, HH), lambda i: (0, 0))),
        out_shape=(_f32((NN, HH)), _f32((2, HH))),
        scratch_shapes=[pltpu.VMEM((2, HH), jnp.float32)],
    )(parts, scaled, dinv_bc, gcn_b[l].reshape(1, HH))

    h, scaled = pl.pallas_call(
        functools.partial(_tc_bn_body, last=last),
        grid=(GB,),
        in_specs=[rows, pl.BlockSpec((2, HH), lambda i: (0, 0)),
                  rows, full_v, full_v, full_w],
        out_specs=(rows, rows),
        out_shape=(_f32((NN, HH)), _f32((NN, HH))),
    )(h_pre, stats, dinv_bc, bn_gamma.reshape(1, HH),
      bn_beta.reshape(1, HH), wnext)
    hs.append(h)

  z_D, z_T = pl.pallas_call(
      _tc_attn_body,
      grid=(GB,),
      in_specs=[rows, rows, rows, full_w, full_w, full_v, full_v,
                full_w, full_v],
      out_specs=(rows, rows),
      out_shape=(_f32((NN, HH)), _f32((NN, HH))),
  )(hs[0], hs[1], hs[2], W_D, W_T, q_D.reshape(1, HH), q_T.reshape(1, HH),
    out_W, out_b.reshape(1, HH))

  zcat = jnp.concatenate([z_D, z_T], axis=0)
  idx_all = jnp.concatenate([
      drug_pos_ids, target_pos_ids + NN, drug_neg_ids, target_neg_ids + NN,
  ]).reshape(NW, 8, CHUNK)
  gathered = sc_take(zcat, idx_all)
  g = gathered.reshape(4, 8192, HH)
  return (g[0], g[1], g[2], g[3])


# speedup vs baseline: 19.3356x; 1.0069x over previous
import functools
import jax
import jax.numpy as jnp
from jax import lax
from jax.experimental import pallas as pl
from jax.experimental.pallas import tpu as pltpu
from jax.experimental.pallas import tpu_sc as plsc

NN = 10000
EE = 320000
HH = 128
NC = 2
NS = 16
NW = NC * NS
CHUNK = 128
NCH = 80
NPAD = 10240
RPT = NPAD // NS
KW = 8
NWIN = NCH // KW
_HIGH = lax.Precision.DEFAULT


@functools.cache
def _sc_kernels():
  mesh = plsc.VectorSubcoreMesh(
      core_axis_name="c", subcore_axis_name="s", num_cores=NC, num_subcores=NS)

  @functools.partial(
      pl.kernel,
      out_type=jax.ShapeDtypeStruct((NC, NPAD, HH), jnp.float32),
      mesh=mesh,
      scratch_types=[
          pltpu.VMEM((NCH, CHUNK), jnp.int32),
          pltpu.VMEM((CHUNK, HH), jnp.float32),
          pltpu.VMEM_SHARED((NPAD, HH), jnp.float32),
      ],
  )
  def sc_degree(dst_hbm, zrow_hbm, ones_hbm, out_hbm, dst_v, ones_v, deg_sp):
    c = lax.axis_index("c")
    s = lax.axis_index("s")
    w = c * NS + s
    pltpu.sync_copy(zrow_hbm, ones_v)
    for k in range(RPT // CHUNK):
      pltpu.sync_copy(ones_v, deg_sp.at[pl.ds((s * 5 + k) * CHUNK, CHUNK)])
    pltpu.sync_copy(dst_hbm.at[w], dst_v)
    pltpu.sync_copy(ones_hbm, ones_v)
    plsc.subcore_barrier()

    @pl.loop(0, NCH)
    def _(j):
      pltpu.sync_copy(ones_v, deg_sp.at[dst_v.at[j]], add=True)

    plsc.subcore_barrier()
    pltpu.sync_copy(deg_sp.at[pl.ds(s * RPT, RPT)],
                    out_hbm.at[c].at[pl.ds(s * RPT, RPT)])

  @functools.partial(
      pl.kernel,
      out_type=jax.ShapeDtypeStruct((NC, NPAD, HH), jnp.float32),
      mesh=mesh,
      scratch_types=[
          pltpu.VMEM((2, KW, CHUNK), jnp.int32),
          pltpu.VMEM((2, KW, CHUNK), jnp.int32),
          pltpu.VMEM((2, CHUNK, HH), jnp.float32),
          pltpu.VMEM_SHARED((NPAD, HH), jnp.float32),
          pltpu.SemaphoreType.DMA,
          pltpu.SemaphoreType.DMA,
          pltpu.SemaphoreType.DMA,
      ],
  )
  def sc_aggregate(scaled_hbm, src_hbm, dst_hbm, zrow_hbm, out_hbm,
                   sidx, didx, rows_v, agg_sp, sem_g, sem_i, sem_s):
    c = lax.axis_index("c")
    s = lax.axis_index("s")
    w = c * NS + s
    pltpu.sync_copy(zrow_hbm, rows_v.at[0])
    for k in range(RPT // CHUNK):
      pltpu.sync_copy(rows_v.at[0], agg_sp.at[pl.ds((s * 5 + k) * CHUNK, CHUNK)])
    plsc.subcore_barrier()

    pltpu.sync_copy(src_hbm.at[w].at[pl.ds(0, KW)], sidx.at[0])
    pltpu.sync_copy(dst_hbm.at[w].at[pl.ds(0, KW)], didx.at[0])

    def gather(k):
      wi, b = divmod(k, KW)
      return pltpu.async_copy(scaled_hbm.at[sidx.at[wi % 2].at[b]],
                              rows_v.at[k % 2], sem_g)

    gd = {0: gather(0)}
    idx_pref = {}
    sd = {}
    for k in range(NCH):
      wi, b = divmod(k, KW)
      if b == 0 and wi + 1 < NWIN:
        nxt = 1 - wi % 2
        idx_pref[wi + 1] = (
            pltpu.async_copy(src_hbm.at[w].at[pl.ds((wi + 1) * KW, KW)],
                             sidx.at[nxt], sem_i),
            pltpu.async_copy(dst_hbm.at[w].at[pl.ds((wi + 1) * KW, KW)],
                             didx.at[nxt], sem_i))
      gd[k].wait()
      sd[k] = pltpu.async_copy(rows_v.at[k % 2],
                               agg_sp.at[didx.at[wi % 2].at[b]],
                               sem_s, add=True)
      if k + 1 < NCH:
        if b + 1 == KW:
          idx_pref[wi + 1][0].wait()
          idx_pref[wi + 1][1].wait()
        if k - 1 >= 0:
          sd[k - 1].wait()
        gd[k + 1] = gather(k + 1)
    sd[NCH - 1].wait()

    plsc.subcore_barrier()
    pltpu.sync_copy(agg_sp.at[pl.ds(s * RPT, RPT)],
                    out_hbm.at[c].at[pl.ds(s * RPT, RPT)])

  @functools.partial(
      pl.kernel,
      out_type=jax.ShapeDtypeStruct((4 * 8192, HH), jnp.float32),
      mesh=mesh,
      scratch_types=[
          pltpu.VMEM((8, CHUNK), jnp.int32),
          pltpu.VMEM((CHUNK, HH), jnp.float32),
          pltpu.SemaphoreType.DMA,
      ],
  )
  def sc_take(table_hbm, idx_hbm, out_hbm, idx_v, rows_v, sem):
    c = lax.axis_index("c")
    s = lax.axis_index("s")
    w = c * NS + s
    pltpu.sync_copy(idx_hbm.at[w], idx_v)

    @pl.loop(0, 8)
    def _(j):
      pltpu.async_copy(table_hbm.at[idx_v.at[j]], rows_v, sem).wait()
      pltpu.sync_copy(rows_v, out_hbm.at[pl.ds((w * 8 + j) * CHUNK, CHUNK)])

  return sc_degree, sc_aggregate, sc_take


RB = 2000
GB = NN // RB


def _tc_prep_body(x_ref, w1_ref, degp_ref, scaled_ref, dinv_ref):
  deg = degp_ref[0][:, 0:1] + degp_ref[1][:, 0:1] + 1.0
  dinv = lax.rsqrt(jnp.maximum(deg, 1.0))
  dinv_bc = jnp.broadcast_to(dinv, (RB, HH))
  hw = lax.dot_general(x_ref[...], w1_ref[...],
                       (((1,), (1,)), ((), ())), precision=_HIGH)
  scaled_ref[...] = hw * dinv_bc
  dinv_ref[...] = dinv_bc


def _tc_stats_body(parts_ref, scaled_ref, dinv_ref, b_ref,
                   h_ref, stats_ref, acc_ref):
  i = pl.program_id(0)
  agg = parts_ref[0] + parts_ref[1] + scaled_ref[...]
  h = jnp.maximum(agg * dinv_ref[...] + b_ref[...], 0.0)
  h_ref[...] = h
  ssum = jnp.sum(h, axis=0, keepdims=True)
  ssq = jnp.sum(jnp.square(h), axis=0, keepdims=True)
  blk = jnp.concatenate([ssum, ssq], axis=0)

  @pl.when(i == 0)
  def _():
    acc_ref[...] = blk

  @pl.when(i > 0)
  def _():
    acc_ref[...] += blk

  @pl.when(i == GB - 1)
  def _():
    stats_ref[...] = acc_ref[...]


def _tc_bn_body(h_ref, stats_ref, dinv_ref, g_ref, be_ref, wnext_ref,
                hbn_ref, snext_ref, *, last):
  mean = stats_ref[0:1] * (1.0 / NN)
  var = stats_ref[1:2] * (1.0 / NN) - jnp.square(mean)
  hbn = ((h_ref[...] - mean) * lax.rsqrt(var + 1e-5) * g_ref[...]
         + be_ref[...])
  hbn_ref[...] = hbn
  if not last:
    hw = lax.dot_general(hbn, wnext_ref[...],
                         (((1,), (1,)), ((), ())), precision=_HIGH)
    snext_ref[...] = hw * dinv_ref[...]


def _tc_attn_body(h0_ref, h1_ref, h2_ref, wd_ref, wt_ref, qd_ref, qt_ref,
                  ow_ref, ob_ref, zd_ref, zt_ref):
  hs = (h0_ref[...], h1_ref[...], h2_ref[...])

  def pool(w, q):
    es = []
    for h in hs:
      hh = lax.dot_general(h, w, (((1,), (1,)), ((), ())), precision=_HIGH)
      hh = jnp.where(hh >= 0.0, hh, 0.01 * hh)
      es.append(jnp.sum(hh * q, axis=1, keepdims=True))
    m = jnp.maximum(jnp.maximum(es[0], es[1]), es[2])
    a = [jnp.exp(e - m) for e in es]
    tot = a[0] + a[1] + a[2]
    pooled = (hs[0] * (a[0] / tot) + hs[1] * (a[1] / tot)
              + hs[2] * (a[2] / tot))
    return lax.dot_general(pooled, ow_ref[...], (((1,), (1,)), ((), ())),
                           precision=_HIGH) + ob_ref[...]

  zd_ref[...] = pool(wd_ref[...], qd_ref[...])
  zt_ref[...] = pool(wt_ref[...], qt_ref[...])


def _f32(shape):
  return jax.ShapeDtypeStruct(shape, jnp.float32)


def kernel(x, edge_index, drug_pos_ids, target_pos_ids, drug_neg_ids,
           target_neg_ids, adjacency_matrix, gcn_W, gcn_b, W_D, W_T, q_D, q_T,
           bn_gamma, bn_beta, out_W, out_b):
  sc_degree, sc_aggregate, sc_take = _sc_kernels()

  ept = EE // NW
  ppt = NCH * CHUNK
  padlen = ppt - ept
  src = edge_index[0].reshape(NW, ept)
  dst = edge_index[1].reshape(NW, ept)
  pad_iota = jnp.arange(NW * padlen, dtype=jnp.int32).reshape(NW, padlen)
  pad_src = pad_iota % NN
  pad_dst = NN + (pad_iota % (NPAD - NN))
  src_p = jnp.concatenate([src, pad_src], 1).reshape(NW, NCH, CHUNK)
  dst_p = jnp.concatenate([dst, pad_dst], 1).reshape(NW, NCH, CHUNK)

  zrow = jnp.zeros((CHUNK, HH), jnp.float32)
  onesrow = jnp.ones((CHUNK, HH), jnp.float32)

  rows = pl.BlockSpec((RB, HH), lambda i: (i, 0))
  full_w = pl.BlockSpec((HH, HH), lambda i: (0, 0))
  full_v = pl.BlockSpec((1, HH), lambda i: (0, 0))

  deg_parts = sc_degree(dst_p, zrow, onesrow)

  scaled, dinv_bc = pl.pallas_call(
      _tc_prep_body,
      grid=(GB,),
      in_specs=[rows, full_w,
                pl.BlockSpec((NC, RB, HH), lambda i: (0, i, 0))],
      out_specs=(rows, rows),
      out_shape=(_f32((NN, HH)), _f32((NN, HH))),
  )(x, gcn_W[0], deg_parts)

  hs = []
  for l in range(3):
    parts = sc_aggregate(scaled, src_p, dst_p, zrow)
    last = l == 2
    wnext = gcn_W[l + 1] if not last else jnp.zeros((HH, HH), jnp.float32)

    h_pre, stats = pl.pallas_call(
        _tc_stats_body,
        grid=(GB,),
        in_specs=[pl.BlockSpec((NC, RB, HH), lambda i: (0, i, 0)),
                  rows, rows, full_v],
        out_specs=(rows, pl.BlockSpec((2, HH), lambda i: (0, 0))),
        out_shape=(_f32((NN, HH)), _f32((2, HH))),
        scratch_shapes=[pltpu.VMEM((2, HH), jnp.float32)],
    )(parts, scaled, dinv_bc, gcn_b[l].reshape(1, HH))

    h, scaled = pl.pallas_call(
        functools.partial(_tc_bn_body, last=last),
        grid=(GB,),
        in_specs=[rows, pl.BlockSpec((2, HH), lambda i: (0, 0)),
                  rows, full_v, full_v, full_w],
        out_specs=(rows, rows),
        out_shape=(_f32((NN, HH)), _f32((NN, HH))),
    )(h_pre, stats, dinv_bc, bn_gamma.reshape(1, HH),
      bn_beta.reshape(1, HH), wnext)
    hs.append(h)

  z_D, z_T = pl.pallas_call(
      _tc_attn_body,
      grid=(GB,),
      in_specs=[rows, rows, rows, full_w, full_w, full_v, full_v,
                full_w, full_v],
      out_specs=(rows, rows),
      out_shape=(_f32((NN, HH)), _f32((NN, HH))),
  )(hs[0], hs[1], hs[2], W_D, W_T, q_D.reshape(1, HH), q_T.reshape(1, HH),
    out_W, out_b.reshape(1, HH))

  zcat = jnp.concatenate([z_D, z_T], axis=0)
  idx_all = jnp.concatenate([
      drug_pos_ids, target_pos_ids + NN, drug_neg_ids, target_neg_ids + NN,
  ]).reshape(NW, 8, CHUNK)
  gathered = sc_take(zcat, idx_all)
  g = gathered.reshape(4, 8192, HH)
  return (g[0], g[1], g[2], g[3])


# speedup vs baseline: 21.2646x; 1.0998x over previous
import functools
import jax
import jax.numpy as jnp
from jax import lax
from jax.experimental import pallas as pl
from jax.experimental.pallas import tpu as pltpu
from jax.experimental.pallas import tpu_sc as plsc

NN = 10000
EE = 320000
HH = 128
NC = 2
NS = 16
NW = NC * NS
CHUNK = 128
NCH = 80
NPAD = 10240
RPT = NPAD // NS
KW = 8
NWIN = NCH // KW
_HIGH = lax.Precision.DEFAULT


@functools.cache
def _sc_kernels():
  mesh = plsc.VectorSubcoreMesh(
      core_axis_name="c", subcore_axis_name="s", num_cores=NC, num_subcores=NS)

  @functools.partial(
      pl.kernel,
      out_type=jax.ShapeDtypeStruct((NC, NPAD // HH, HH), jnp.float32),
      mesh=mesh,
      compiler_params=pltpu.CompilerParams(needs_layout_passes=False),
      scratch_types=[
          pltpu.VMEM((NCH, CHUNK), jnp.int32),
          pltpu.VMEM((NPAD,), jnp.float32),
          pltpu.VMEM((NS, 1024), jnp.float32),
          pltpu.VMEM((8, HH), jnp.float32),
          pltpu.VMEM_SHARED((NS, NPAD), jnp.float32),
      ],
  )
  def sc_degree(dst_hbm, out_hbm, dst_v, hist_v, red_v, res_v, part_sp):
    c = lax.axis_index("c")
    s = lax.axis_index("s")
    w = c * NS + s
    pltpu.sync_copy(dst_hbm.at[w], dst_v)
    zero = jnp.zeros((16,), jnp.float32)
    for t in range(NPAD // 16):
      hist_v[pl.ds(t * 16, 16)] = zero
    ones = jnp.ones((16,), jnp.float32)
    for j in range(NCH):
      for q in range(CHUNK // 16):
        iv = dst_v[j, pl.ds(q * 16, 16)]
        plsc.addupdate_scatter(hist_v, [iv], ones)
    pltpu.sync_copy(hist_v, part_sp.at[s])
    plsc.subcore_barrier()

    @pl.when(s < NPAD // 1024)
    def _():
      for r in range(NS):
        pltpu.sync_copy(part_sp.at[r].at[pl.ds(s * 1024, 1024)], red_v.at[r])
      for t in range(1024 // 16):
        acc = red_v[0, pl.ds(t * 16, 16)]
        for r in range(1, NS):
          acc = acc + red_v[r, pl.ds(t * 16, 16)]
        res_v[t // 8, pl.ds((t % 8) * 16, 16)] = acc
      pltpu.sync_copy(res_v, out_hbm.at[c].at[pl.ds(s * 8, 8)])

  @functools.partial(
      pl.kernel,
      out_type=jax.ShapeDtypeStruct((NC, NPAD, HH), jnp.float32),
      mesh=mesh,
      scratch_types=[
          pltpu.VMEM((2, KW, CHUNK), jnp.int32),
          pltpu.VMEM((2, KW, CHUNK), jnp.int32),
          pltpu.VMEM((2, CHUNK, HH), jnp.float32),
          pltpu.VMEM_SHARED((NPAD, HH), jnp.float32),
          pltpu.SemaphoreType.DMA,
          pltpu.SemaphoreType.DMA,
          pltpu.SemaphoreType.DMA,
      ],
  )
  def sc_aggregate(scaled_hbm, src_hbm, dst_hbm, zrow_hbm, out_hbm,
                   sidx, didx, rows_v, agg_sp, sem_g, sem_i, sem_s):
    c = lax.axis_index("c")
    s = lax.axis_index("s")
    w = c * NS + s
    pltpu.sync_copy(zrow_hbm, rows_v.at[0])
    for k in range(RPT // CHUNK):
      pltpu.sync_copy(rows_v.at[0], agg_sp.at[pl.ds((s * 5 + k) * CHUNK, CHUNK)])
    plsc.subcore_barrier()

    pltpu.sync_copy(src_hbm.at[w].at[pl.ds(0, KW)], sidx.at[0])
    pltpu.sync_copy(dst_hbm.at[w].at[pl.ds(0, KW)], didx.at[0])

    def gather(k):
      wi, b = divmod(k, KW)
      return pltpu.async_copy(scaled_hbm.at[sidx.at[wi % 2].at[b]],
                              rows_v.at[k % 2], sem_g)

    gd = {0: gather(0)}
    idx_pref = {}
    sd = {}
    for k in range(NCH):
      wi, b = divmod(k, KW)
      if b == 0 and wi + 1 < NWIN:
        nxt = 1 - wi % 2
        idx_pref[wi + 1] = (
            pltpu.async_copy(src_hbm.at[w].at[pl.ds((wi + 1) * KW, KW)],
                             sidx.at[nxt], sem_i),
            pltpu.async_copy(dst_hbm.at[w].at[pl.ds((wi + 1) * KW, KW)],
                             didx.at[nxt], sem_i))
      gd[k].wait()
      sd[k] = pltpu.async_copy(rows_v.at[k % 2],
                               agg_sp.at[didx.at[wi % 2].at[b]],
                               sem_s, add=True)
      if k + 1 < NCH:
        if b + 1 == KW:
          idx_pref[wi + 1][0].wait()
          idx_pref[wi + 1][1].wait()
        if k - 1 >= 0:
          sd[k - 1].wait()
        gd[k + 1] = gather(k + 1)
    sd[NCH - 1].wait()

    plsc.subcore_barrier()
    pltpu.sync_copy(agg_sp.at[pl.ds(s * RPT, RPT)],
                    out_hbm.at[c].at[pl.ds(s * RPT, RPT)])

  @functools.partial(
      pl.kernel,
      out_type=jax.ShapeDtypeStruct((4 * 8192, HH), jnp.float32),
      mesh=mesh,
      scratch_types=[
          pltpu.VMEM((8, CHUNK), jnp.int32),
          pltpu.VMEM((CHUNK, HH), jnp.float32),
          pltpu.SemaphoreType.DMA,
      ],
  )
  def sc_take(table_hbm, idx_hbm, out_hbm, idx_v, rows_v, sem):
    c = lax.axis_index("c")
    s = lax.axis_index("s")
    w = c * NS + s
    pltpu.sync_copy(idx_hbm.at[w], idx_v)

    @pl.loop(0, 8)
    def _(j):
      pltpu.async_copy(table_hbm.at[idx_v.at[j]], rows_v, sem).wait()
      pltpu.sync_copy(rows_v, out_hbm.at[pl.ds((w * 8 + j) * CHUNK, CHUNK)])

  return sc_degree, sc_aggregate, sc_take


RB = 2000
GB = NN // RB


def _tc_prep_body(x_ref, w1_ref, degp_ref, scaled_ref, dinv_ref):
  i = pl.program_id(0)
  d80 = degp_ref[0] + degp_ref[1] + 1.0
  p0 = i * RB
  rid = jax.lax.broadcasted_iota(jnp.int32, (RB, NPAD // HH), 0) + p0
  wrow = (rid // HH == jax.lax.broadcasted_iota(
      jnp.int32, (RB, NPAD // HH), 1)).astype(jnp.float32)
  x80 = lax.dot_general(wrow, d80, (((1,), (0,)), ((), ())),
                        precision=lax.Precision.HIGHEST)
  lid = jax.lax.broadcasted_iota(jnp.int32, (RB, HH), 0) + p0
  lmask = (lid % HH == jax.lax.broadcasted_iota(jnp.int32, (RB, HH), 1))
  deg = jnp.sum(jnp.where(lmask, x80, 0.0), axis=1, keepdims=True)
  dinv = lax.rsqrt(jnp.maximum(deg, 1.0))
  dinv_bc = jnp.broadcast_to(dinv, (RB, HH))
  hw = lax.dot_general(x_ref[...], w1_ref[...],
                       (((1,), (1,)), ((), ())), precision=_HIGH)
  scaled_ref[...] = hw * dinv_bc
  dinv_ref[...] = dinv_bc


def _tc_stats_body(parts_ref, scaled_ref, dinv_ref, b_ref,
                   h_ref, stats_ref, acc_ref):
  i = pl.program_id(0)
  agg = parts_ref[0] + parts_ref[1] + scaled_ref[...]
  h = jnp.maximum(agg * dinv_ref[...] + b_ref[...], 0.0)
  h_ref[...] = h
  ssum = jnp.sum(h, axis=0, keepdims=True)
  ssq = jnp.sum(jnp.square(h), axis=0, keepdims=True)
  blk = jnp.concatenate([ssum, ssq], axis=0)

  @pl.when(i == 0)
  def _():
    acc_ref[...] = blk

  @pl.when(i > 0)
  def _():
    acc_ref[...] += blk

  @pl.when(i == GB - 1)
  def _():
    stats_ref[...] = acc_ref[...]


def _tc_bn_body(h_ref, stats_ref, dinv_ref, g_ref, be_ref, wnext_ref,
                hbn_ref, snext_ref, *, last):
  mean = stats_ref[0:1] * (1.0 / NN)
  var = stats_ref[1:2] * (1.0 / NN) - jnp.square(mean)
  hbn = ((h_ref[...] - mean) * lax.rsqrt(var + 1e-5) * g_ref[...]
         + be_ref[...])
  hbn_ref[...] = hbn
  if not last:
    hw = lax.dot_general(hbn, wnext_ref[...],
                         (((1,), (1,)), ((), ())), precision=_HIGH)
    snext_ref[...] = hw * dinv_ref[...]


def _tc_attn_body(h0_ref, h1_ref, h2_ref, wd_ref, wt_ref, qd_ref, qt_ref,
                  ow_ref, ob_ref, zd_ref, zt_ref):
  hs = (h0_ref[...], h1_ref[...], h2_ref[...])

  def pool(w, q):
    es = []
    for h in hs:
      hh = lax.dot_general(h, w, (((1,), (1,)), ((), ())), precision=_HIGH)
      hh = jnp.where(hh >= 0.0, hh, 0.01 * hh)
      es.append(jnp.sum(hh * q, axis=1, keepdims=True))
    m = jnp.maximum(jnp.maximum(es[0], es[1]), es[2])
    a = [jnp.exp(e - m) for e in es]
    tot = a[0] + a[1] + a[2]
    pooled = (hs[0] * (a[0] / tot) + hs[1] * (a[1] / tot)
              + hs[2] * (a[2] / tot))
    return lax.dot_general(pooled, ow_ref[...], (((1,), (1,)), ((), ())),
                           precision=_HIGH) + ob_ref[...]

  zd_ref[...] = pool(wd_ref[...], qd_ref[...])
  zt_ref[...] = pool(wt_ref[...], qt_ref[...])


def _f32(shape):
  return jax.ShapeDtypeStruct(shape, jnp.float32)


def kernel(x, edge_index, drug_pos_ids, target_pos_ids, drug_neg_ids,
           target_neg_ids, adjacency_matrix, gcn_W, gcn_b, W_D, W_T, q_D, q_T,
           bn_gamma, bn_beta, out_W, out_b):
  sc_degree, sc_aggregate, sc_take = _sc_kernels()

  ept = EE // NW
  ppt = NCH * CHUNK
  padlen = ppt - ept
  src = edge_index[0].reshape(NW, ept)
  dst = edge_index[1].reshape(NW, ept)
  pad_iota = jnp.arange(NW * padlen, dtype=jnp.int32).reshape(NW, padlen)
  pad_src = pad_iota % NN
  pad_dst = NN + (pad_iota % (NPAD - NN))
  src_p = jnp.concatenate([src, pad_src], 1).reshape(NW, NCH, CHUNK)
  dst_p = jnp.concatenate([dst, pad_dst], 1).reshape(NW, NCH, CHUNK)

  zrow = jnp.zeros((CHUNK, HH), jnp.float32)

  rows = pl.BlockSpec((RB, HH), lambda i: (i, 0))
  full_w = pl.BlockSpec((HH, HH), lambda i: (0, 0))
  full_v = pl.BlockSpec((1, HH), lambda i: (0, 0))

  deg_parts = sc_degree(dst_p)

  scaled, dinv_bc = pl.pallas_call(
      _tc_prep_body,
      grid=(GB,),
      in_specs=[rows, full_w,
                pl.BlockSpec((NC, NPAD // HH, HH), lambda i: (0, 0, 0))],
      out_specs=(rows, rows),
      out_shape=(_f32((NN, HH)), _f32((NN, HH))),
  )(x, gcn_W[0], deg_parts)

  hs = []
  for l in range(3):
    parts = sc_aggregate(scaled, src_p, dst_p, zrow)
    last = l == 2
    wnext = gcn_W[l + 1] if not last else jnp.zeros((HH, HH), jnp.float32)

    h_pre, stats = pl.pallas_call(
        _tc_stats_body,
        grid=(GB,),
        in_specs=[pl.BlockSpec((NC, RB, HH), lambda i: (0, i, 0)),
                  rows, rows, full_v],
        out_specs=(rows, pl.BlockSpec((2, HH), lambda i: (0, 0))),
        out_shape=(_f32((NN, HH)), _f32((2, HH))),
        scratch_shapes=[pltpu.VMEM((2, HH), jnp.float32)],
    )(parts, scaled, dinv_bc, gcn_b[l].reshape(1, HH))

    h, scaled = pl.pallas_call(
        functools.partial(_tc_bn_body, last=last),
        grid=(GB,),
        in_specs=[rows, pl.BlockSpec((2, HH), lambda i: (0, 0)),
                  rows, full_v, full_v, full_w],
        out_specs=(rows, rows),
        out_shape=(_f32((NN, HH)), _f32((NN, HH))),
    )(h_pre, stats, dinv_bc, bn_gamma.reshape(1, HH),
      bn_beta.reshape(1, HH), wnext)
    hs.append(h)

  z_D, z_T = pl.pallas_call(
      _tc_attn_body,
      grid=(GB,),
      in_specs=[rows, rows, rows, full_w, full_w, full_v, full_v,
                full_w, full_v],
      out_specs=(rows, rows),
      out_shape=(_f32((NN, HH)), _f32((NN, HH))),
  )(hs[0], hs[1], hs[2], W_D, W_T, q_D.reshape(1, HH), q_T.reshape(1, HH),
    out_W, out_b.reshape(1, HH))

  zcat = jnp.concatenate([z_D, z_T], axis=0)
  idx_all = jnp.concatenate([
      drug_pos_ids, target_pos_ids + NN, drug_neg_ids, target_neg_ids + NN,
  ]).reshape(NW, 8, CHUNK)
  gathered = sc_take(zcat, idx_all)
  g = gathered.reshape(4, 8192, HH)
  return (g[0], g[1], g[2], g[3])
